# Initial kernel scaffold; baseline (speedup 1.0000x reference)
import jax
import jax.numpy as jnp
from jax.experimental import pallas as pl


def kernel(dec_x, dec_pc, enc_x, enc_pc, W_pre1, b_pre1, W_pre2, b_pre2, Wq, bq, Wk, bk, Wv, bv, pos_W1, pos_b1, pos_g, pos_be, pos_W2, pos_b2, attn_W1, attn_b1, attn_g, attn_be, attn_W2, attn_b2, W_post1, b_post1, W_post2, b_post2):
    raise NotImplementedError("write your pallas kernel here")


# speedup vs baseline: 17.2339x; 17.2339x over previous
import functools

import jax
import jax.numpy as jnp
from jax import lax
from jax.experimental import pallas as pl
from jax.experimental.pallas import tpu as pltpu
from jax.experimental.pallas import tpu_sc as plsc

B, N1, N2 = 16, 2048, 1024
M = N1 + N2
IN1, IN2, DIM = 256, 256, 64
PH, AH, KNN = 32, 32, 16
EPS = 1e-5
CNT = float(B * N1 * KNN)

NC, NS = 2, 16
NW = NC * NS

_f32 = jnp.float32
_i32 = jnp.int32


def _t1_body(x_ref, pc_ref, Wp_ref, bp_ref, Wq_ref, bq_ref, WA_ref, Wp1_ref,
             dxp_ref, qA_ref, Pq_ref):
    x = x_ref[0]
    dxp = jnp.dot(x, Wp_ref[...], preferred_element_type=_f32) + bp_ref[...]
    q = jnp.dot(dxp, Wq_ref[...], preferred_element_type=_f32) + bq_ref[...]
    qA = jnp.dot(q, WA_ref[...], preferred_element_type=_f32)
    pc = pc_ref[0]
    Pq = (pc[:, 0:1] * Wp1_ref[0:1, :] + pc[:, 1:2] * Wp1_ref[1:2, :]
          + pc[:, 2:3] * Wp1_ref[2:3, :])
    dxp_ref[0] = dxp
    qA_ref[0] = qA
    Pq_ref[0] = Pq


def _t2_body(x_ref, Wp_ref, bp_ref, out_ref):
    out_ref[0] = (jnp.dot(x_ref[0], Wp_ref[...], preferred_element_type=_f32)
                  + bp_ref[...])


def _fps_body(pc_ref, idx_ref):
    X = pc_ref[0]
    Y = pc_ref[1]
    Z = pc_ref[2]
    col = lax.broadcasted_iota(_i32, (B, M), 1)
    coln1 = lax.broadcasted_iota(_i32, (B, N1), 1)
    boff = lax.broadcasted_iota(_i32, (B, 1), 0) * M

    def step(i, carry):
        dists, far, gidx = carry
        gidx = jnp.where(coln1 == i, far + boff, gidx)
        mask = col == far
        xf = jnp.sum(jnp.where(mask, X, 0.0), axis=1, keepdims=True)
        yf = jnp.sum(jnp.where(mask, Y, 0.0), axis=1, keepdims=True)
        zf = jnp.sum(jnp.where(mask, Z, 0.0), axis=1, keepdims=True)
        dx = X - xf
        dy = Y - yf
        dz = Z - zf
        d = dx * dx + dy * dy + dz * dz
        dists = jnp.minimum(dists, d)
        m = jnp.max(dists, axis=1, keepdims=True)
        far = jnp.min(jnp.where(dists == m, col, M), axis=1, keepdims=True)
        return dists, far, gidx

    _, _, gidx = lax.fori_loop(
        0, N1, step,
        (jnp.full((B, M), 1e10, _f32), jnp.zeros((B, 1), _i32),
         jnp.zeros((B, N1), _i32)))
    idx_ref[...] = gidx


def _t5a_body(x_ref, Wv_ref, bv_ref, Wk_ref, bk_ref, WA_ref, Wp1_ref,
              out_ref):
    x = x_ref[:, :DIM]
    v = jnp.dot(x, Wv_ref[...], preferred_element_type=_f32) + bv_ref[...]
    k = jnp.dot(x, Wk_ref[...], preferred_element_type=_f32) + bk_ref[...]
    kA = jnp.dot(k, WA_ref[...], preferred_element_type=_f32)
    pc = x_ref[:, DIM:DIM + 3]
    P = (pc[:, 0:1] * Wp1_ref[0:1, :] + pc[:, 1:2] * Wp1_ref[1:2, :]
         + pc[:, 2:3] * Wp1_ref[2:3, :])
    out_ref[...] = jnp.concatenate([v, kA, P], axis=1)


def _t5b_body(qpc_ref, spc_ref, out_ref, *, Qk):
    qx = qpc_ref[0, :, 0:1]
    qy = qpc_ref[0, :, 1:2]
    qz = qpc_ref[0, :, 2:3]
    sx = spc_ref[0, 0:1, :]
    sy = spc_ref[0, 1:2, :]
    sz = spc_ref[0, 2:3, :]
    dx = qx - sx
    dy = qy - sy
    dz = qz - sz
    d = dx * dx + dy * dy + dz * dz
    col = lax.broadcasted_iota(_i32, (Qk, N1), 1)
    boff = pl.program_id(0) * N1
    for r in range(KNN):
        m = jnp.min(d, axis=1, keepdims=True)
        idx = jnp.min(jnp.where(d == m, col, 1 << 30), axis=1, keepdims=True)
        out_ref[0, :, pl.ds(r, 1)] = idx + boff
        d = jnp.where(col == idx, jnp.float32(3e38), d)


def _sc_gather(table, idx, width, rows_per_worker):
    R = idx.shape[0]
    nchunk = rows_per_worker // 128
    idx2 = idx.reshape(R // 128, 128)
    mesh = plsc.VectorSubcoreMesh(core_axis_name="c", subcore_axis_name="s")

    @functools.partial(
        pl.kernel, mesh=mesh,
        out_type=jax.ShapeDtypeStruct((R, width), _f32),
        scratch_types=[
            pltpu.VMEM((nchunk, 128), _i32),
            pltpu.VMEM((128, width), _f32),
            pltpu.SemaphoreType.DMA,
        ],
    )
    def k(table_hbm, idx_hbm, out_hbm, idx_v, rows_v, sem):
        wid = lax.axis_index("s") * NC + lax.axis_index("c")
        base = wid * nchunk
        pltpu.sync_copy(idx_hbm.at[pl.ds(base, nchunk)], idx_v)
        for c in range(nchunk):
            pltpu.async_copy(table_hbm.at[idx_v.at[c]], rows_v, sem).wait()
            pltpu.sync_copy(rows_v,
                            out_hbm.at[pl.ds((base + c) * 128, 128)])

    return k(table, idx2)


def _t6_body(g_ref, Pq_ref, b1_ref, out_ref):
    pp = g_ref[0][:, :, DIM + AH:]
    pos_pre = pp - Pq_ref[0][:, None, :] + b1_ref[...]
    f = pos_pre.reshape(-1, PH)
    out_ref[0, 0, 0:1, :] = jnp.sum(f, axis=0, keepdims=True)
    out_ref[0, 0, 1:2, :] = jnp.sum(f * f, axis=0, keepdims=True)


def _stats_final_body(st_ref, g_ref, be_ref, out_ref):
    S = jnp.sum(st_ref[:, :, 0:1, :], axis=(0, 1))
    SS = jnp.sum(st_ref[:, :, 1:2, :], axis=(0, 1))
    mean = S / CNT
    var = SS / CNT - mean * mean
    scale = g_ref[...] * lax.rsqrt(var + EPS)
    out_ref[0:1, :] = scale
    out_ref[1:2, :] = be_ref[...] - mean * scale


def _pos_hidden(pp, Pq, b1, s1):
    pos_pre = pp - Pq[:, None, :] + b1
    return jnp.maximum(pos_pre * s1[0:1, :] + s1[1:2, :], 0.0)


def _t7_body(g_ref, Pq_ref, qA_ref, s1_ref, b1_ref, W2_ref, b2_ref,
             WA_ref, ab1_ref, out_ref):
    g = g_ref[0]
    kA = g[:, :, DIM:DIM + AH]
    posH = _pos_hidden(g[:, :, DIM + AH:], Pq_ref[0], b1_ref[...], s1_ref[...])
    W2A = jnp.dot(W2_ref[...], WA_ref[...], preferred_element_type=_f32)
    b2A = jnp.dot(b2_ref[...], WA_ref[...], preferred_element_type=_f32)
    Q = posH.shape[0]
    posA = jnp.dot(posH.reshape(-1, PH), W2A,
                   preferred_element_type=_f32).reshape(Q, KNN, AH)
    attn_pre = (kA - qA_ref[0][:, None, :] + posA + b2A
                + ab1_ref[...])
    f = attn_pre.reshape(-1, AH)
    out_ref[0, 0, 0:1, :] = jnp.sum(f, axis=0, keepdims=True)
    out_ref[0, 0, 1:2, :] = jnp.sum(f * f, axis=0, keepdims=True)


def _t8_body(g_ref, Pq_ref, qA_ref, s1_ref, s2_ref,
             resid_ref, b1_ref, W2_ref, b2_ref, WA_ref, ab1_ref,
             aW2_ref, ab2_ref, Wo1_ref, bo1_ref, Wo2_ref, bo2_ref,
             dec_ref, enc_ref):
    g = g_ref[0]
    vv = g[:, :, :DIM]
    kA = g[:, :, DIM:DIM + AH]
    posH = _pos_hidden(g[:, :, DIM + AH:], Pq_ref[0], b1_ref[...], s1_ref[...])
    Q = posH.shape[0]
    posHf = posH.reshape(-1, PH)
    pos = (jnp.dot(posHf, W2_ref[...], preferred_element_type=_f32)
           .reshape(Q, KNN, DIM) + b2_ref[...])
    W2A = jnp.dot(W2_ref[...], WA_ref[...], preferred_element_type=_f32)
    b2A = jnp.dot(b2_ref[...], WA_ref[...], preferred_element_type=_f32)
    posA = jnp.dot(posHf, W2A, preferred_element_type=_f32).reshape(Q, KNN, AH)
    attn_pre = (kA - qA_ref[0][:, None, :] + posA + b2A
                + ab1_ref[...])
    h2 = jnp.maximum(attn_pre * s2_ref[0:1, :] + s2_ref[1:2, :], 0.0)
    attn = (jnp.dot(h2.reshape(-1, AH), aW2_ref[...],
                    preferred_element_type=_f32).reshape(Q, KNN, DIM)
            + ab2_ref[...])
    mx = jnp.max(attn, axis=1, keepdims=True)
    e = jnp.exp(attn - mx)
    a = e / jnp.sum(e, axis=1, keepdims=True)
    o = jnp.sum((vv + pos) * a, axis=1)
    dec_ref[0] = (jnp.dot(o, Wo1_ref[...], preferred_element_type=_f32)
                  + bo1_ref[...] + resid_ref[0])
    enc_ref[0] = (jnp.dot(o, Wo2_ref[...], preferred_element_type=_f32)
                  + bo2_ref[...])


def _full(a):
    return pl.BlockSpec(a.shape, lambda *_: (0,) * a.ndim)


def kernel(dec_x, dec_pc, enc_x, enc_pc, W_pre1, b_pre1, W_pre2, b_pre2,
           Wq, bq, Wk, bk, Wv, bv, pos_W1, pos_b1, pos_g, pos_be,
           pos_W2, pos_b2, attn_W1, attn_b1, attn_g, attn_be,
           attn_W2, attn_b2, W_post1, b_post1, W_post2, b_post2):
    r2 = lambda v: v.reshape(1, -1)
    b_pre1r, bqr, bkr, bvr = r2(b_pre1), r2(bq), r2(bk), r2(bv)
    b_pre2r, pos_b1r, pos_b2r = r2(b_pre2), r2(pos_b1), r2(pos_b2)
    attn_b1r, attn_b2r = r2(attn_b1), r2(attn_b2)
    b_post1r, b_post2r = r2(b_post1), r2(b_post2)
    pos_gr, pos_ber, attn_gr, attn_ber = (r2(pos_g), r2(pos_be),
                                          r2(attn_g), r2(attn_be))

    Q1 = 256
    dxp, qA, Pq = pl.pallas_call(
        _t1_body,
        grid=(B, N1 // Q1),
        in_specs=[
            pl.BlockSpec((1, Q1, IN1), lambda b, n: (b, n, 0)),
            pl.BlockSpec((1, Q1, 3), lambda b, n: (b, n, 0)),
            _full(W_pre1), _full(b_pre1r), _full(Wq), _full(bqr),
            _full(attn_W1), _full(pos_W1),
        ],
        out_specs=[
            pl.BlockSpec((1, Q1, DIM), lambda b, n: (b, n, 0)),
            pl.BlockSpec((1, Q1, AH), lambda b, n: (b, n, 0)),
            pl.BlockSpec((1, Q1, PH), lambda b, n: (b, n, 0)),
        ],
        out_shape=[
            jax.ShapeDtypeStruct((B, N1, DIM), _f32),
            jax.ShapeDtypeStruct((B, N1, AH), _f32),
            jax.ShapeDtypeStruct((B, N1, PH), _f32),
        ],
    )(dec_x, dec_pc, W_pre1, b_pre1r, Wq, bqr, attn_W1, pos_W1)

    Q2 = 256
    exp_ = pl.pallas_call(
        _t2_body,
        grid=(B, N2 // Q2),
        in_specs=[
            pl.BlockSpec((1, Q2, IN2), lambda b, n: (b, n, 0)),
            _full(W_pre2), _full(b_pre2r),
        ],
        out_specs=pl.BlockSpec((1, Q2, DIM), lambda b, n: (b, n, 0)),
        out_shape=jax.ShapeDtypeStruct((B, N2, DIM), _f32),
    )(enc_x, W_pre2, b_pre2r)

    pc_cat = jnp.concatenate([dec_pc, enc_pc], axis=1)
    xflat = jnp.concatenate(
        [jnp.concatenate([dxp, exp_], axis=1).reshape(B * M, DIM),
         pc_cat.reshape(B * M, 3)], axis=1)
    xflat = jnp.pad(xflat, ((0, 0), (0, 128 - DIM - 3)))
    pc_t = jnp.transpose(pc_cat, (2, 0, 1))

    fps_gidx = pl.pallas_call(
        _fps_body,
        in_specs=[_full(pc_t)],
        out_specs=pl.BlockSpec((B, N1), lambda: (0, 0)),
        out_shape=jax.ShapeDtypeStruct((B, N1), _i32),
    )(pc_t)

    x_s = _sc_gather(xflat, fps_gidx.reshape(-1), 128, (B * N1) // NW)

    TW = DIM + AH + PH
    Qa = 1024
    tbl = pl.pallas_call(
        _t5a_body,
        grid=((B * N1) // Qa,),
        in_specs=[
            pl.BlockSpec((Qa, 128), lambda i: (i, 0)),
            _full(Wv), _full(bvr), _full(Wk), _full(bkr),
            _full(attn_W1), _full(pos_W1),
        ],
        out_specs=pl.BlockSpec((Qa, TW), lambda i: (i, 0)),
        out_shape=jax.ShapeDtypeStruct((B * N1, TW), _f32),
    )(x_s, Wv, bvr, Wk, bkr, attn_W1, pos_W1)

    Qk = 128
    pcs_b = jnp.transpose(
        x_s[:, DIM:DIM + 3].reshape(B, N1, 3), (0, 2, 1))
    knn_gidx = pl.pallas_call(
        functools.partial(_t5b_body, Qk=Qk),
        grid=(B, N1 // Qk),
        in_specs=[
            pl.BlockSpec((1, Qk, 3), lambda b, n: (b, n, 0)),
            pl.BlockSpec((1, 3, N1), lambda b, n: (b, 0, 0)),
        ],
        out_specs=pl.BlockSpec((1, Qk, KNN), lambda b, n: (b, n, 0)),
        out_shape=jax.ShapeDtypeStruct((B, N1, KNN), _i32),
    )(dec_pc, pcs_b)

    G = _sc_gather(tbl, knn_gidx.reshape(-1), TW, (B * N1 * KNN) // NW)
    G4 = G.reshape(B, N1, KNN, TW)

    Qs = 128
    nq = N1 // Qs
    st1 = pl.pallas_call(
        _t6_body,
        grid=(B, nq),
        in_specs=[
            pl.BlockSpec((1, Qs, KNN, TW), lambda b, n: (b, n, 0, 0)),
            pl.BlockSpec((1, Qs, PH), lambda b, n: (b, n, 0)),
            _full(pos_b1r),
        ],
        out_specs=pl.BlockSpec((1, 1, 2, PH), lambda b, n: (b, n, 0, 0)),
        out_shape=jax.ShapeDtypeStruct((B, nq, 2, PH), _f32),
    )(G4, Pq, pos_b1r)
    s1 = pl.pallas_call(
        _stats_final_body,
        in_specs=[_full(st1), _full(pos_gr), _full(pos_ber)],
        out_specs=pl.BlockSpec((2, PH), lambda: (0, 0)),
        out_shape=jax.ShapeDtypeStruct((2, PH), _f32),
    )(st1, pos_gr, pos_ber)

    st2 = pl.pallas_call(
        _t7_body,
        grid=(B, nq),
        in_specs=[
            pl.BlockSpec((1, Qs, KNN, TW), lambda b, n: (b, n, 0, 0)),
            pl.BlockSpec((1, Qs, PH), lambda b, n: (b, n, 0)),
            pl.BlockSpec((1, Qs, AH), lambda b, n: (b, n, 0)),
            _full(s1), _full(pos_b1r), _full(pos_W2), _full(pos_b2r),
            _full(attn_W1), _full(attn_b1r),
        ],
        out_specs=pl.BlockSpec((1, 1, 2, AH), lambda b, n: (b, n, 0, 0)),
        out_shape=jax.ShapeDtypeStruct((B, nq, 2, AH), _f32),
    )(G4, Pq, qA, s1, pos_b1r, pos_W2, pos_b2r, attn_W1, attn_b1r)
    s2 = pl.pallas_call(
        _stats_final_body,
        in_specs=[_full(st2), _full(attn_gr), _full(attn_ber)],
        out_specs=pl.BlockSpec((2, AH), lambda: (0, 0)),
        out_shape=jax.ShapeDtypeStruct((2, AH), _f32),
    )(st2, attn_gr, attn_ber)

    Qf = 128
    dec_out, enc_out = pl.pallas_call(
        _t8_body,
        grid=(B, N1 // Qf),
        in_specs=[
            pl.BlockSpec((1, Qf, KNN, TW), lambda b, n: (b, n, 0, 0)),
            pl.BlockSpec((1, Qf, PH), lambda b, n: (b, n, 0)),
            pl.BlockSpec((1, Qf, AH), lambda b, n: (b, n, 0)),
            _full(s1), _full(s2),
            pl.BlockSpec((1, Qf, IN1), lambda b, n: (b, n, 0)),
            _full(pos_b1r), _full(pos_W2), _full(pos_b2r), _full(attn_W1),
            _full(attn_b1r), _full(attn_W2), _full(attn_b2r),
            _full(W_post1), _full(b_post1r), _full(W_post2), _full(b_post2r),
        ],
        out_specs=[
            pl.BlockSpec((1, Qf, IN1), lambda b, n: (b, n, 0)),
            pl.BlockSpec((1, Qf, IN2), lambda b, n: (b, n, 0)),
        ],
        out_shape=[
            jax.ShapeDtypeStruct((B, N1, IN1), _f32),
            jax.ShapeDtypeStruct((B, N1, IN2), _f32),
        ],
    )(G4, Pq, qA, s1, s2, dec_x, pos_b1r, pos_W2, pos_b2r,
      attn_W1, attn_b1r, attn_W2, attn_b2r, W_post1, b_post1r,
      W_post2, b_post2r)

    return (dec_out, dec_pc, enc_out, enc_pc)


# speedup vs baseline: 22.4235x; 1.3011x over previous
import functools

import jax
import jax.numpy as jnp
from jax import lax
from jax.experimental import pallas as pl
from jax.experimental.pallas import tpu as pltpu
from jax.experimental.pallas import tpu_sc as plsc

B, N1, N2 = 16, 2048, 1024
M = N1 + N2
IN1, IN2, DIM = 256, 256, 64
PH, AH, KNN = 32, 32, 16
EPS = 1e-5
CNT = float(B * N1 * KNN)

NC, NS = 2, 16
NW = NC * NS

_f32 = jnp.float32
_i32 = jnp.int32


def _t1_body(x_ref, pc_ref, Wp_ref, bp_ref, Wq_ref, bq_ref, WA_ref, Wp1_ref,
             dxp_ref, qA_ref, Pq_ref):
    x = x_ref[0]
    dxp = jnp.dot(x, Wp_ref[...], preferred_element_type=_f32) + bp_ref[...]
    q = jnp.dot(dxp, Wq_ref[...], preferred_element_type=_f32) + bq_ref[...]
    qA = jnp.dot(q, WA_ref[...], preferred_element_type=_f32)
    pc = pc_ref[0]
    Pq = (pc[:, 0:1] * Wp1_ref[0:1, :] + pc[:, 1:2] * Wp1_ref[1:2, :]
          + pc[:, 2:3] * Wp1_ref[2:3, :])
    dxp_ref[0] = dxp
    qA_ref[0] = qA
    Pq_ref[0] = Pq


def _t2_body(x_ref, Wp_ref, bp_ref, out_ref):
    out_ref[0] = (jnp.dot(x_ref[0], Wp_ref[...], preferred_element_type=_f32)
                  + bp_ref[...])


def _fps_body(pc_ref, idx_ref):
    X = pc_ref[0]
    Y = pc_ref[1]
    Z = pc_ref[2]
    col = lax.broadcasted_iota(_i32, (B, M), 1)
    coln1 = lax.broadcasted_iota(_i32, (B, N1), 1)
    boff = lax.broadcasted_iota(_i32, (B, 1), 0) * M

    def step(i, carry):
        dists, far, gidx = carry
        gidx = jnp.where(coln1 == i, far + boff, gidx)
        mask = col == far
        xf = jnp.sum(jnp.where(mask, X, 0.0), axis=1, keepdims=True)
        yf = jnp.sum(jnp.where(mask, Y, 0.0), axis=1, keepdims=True)
        zf = jnp.sum(jnp.where(mask, Z, 0.0), axis=1, keepdims=True)
        dx = X - xf
        dy = Y - yf
        dz = Z - zf
        d = dx * dx + dy * dy + dz * dz
        dists = jnp.minimum(dists, d)
        m = jnp.max(dists, axis=1, keepdims=True)
        far = jnp.min(jnp.where(dists == m, col, M), axis=1, keepdims=True)
        return dists, far, gidx

    _, _, gidx = lax.fori_loop(
        0, N1, step,
        (jnp.full((B, M), 1e10, _f32), jnp.zeros((B, 1), _i32),
         jnp.zeros((B, N1), _i32)))
    idx_ref[...] = gidx


def _t5a_body(x_ref, Wv_ref, bv_ref, Wk_ref, bk_ref, WA_ref, Wp1_ref,
              out_ref):
    x = x_ref[:, :DIM]
    v = jnp.dot(x, Wv_ref[...], preferred_element_type=_f32) + bv_ref[...]
    k = jnp.dot(x, Wk_ref[...], preferred_element_type=_f32) + bk_ref[...]
    kA = jnp.dot(k, WA_ref[...], preferred_element_type=_f32)
    pc = x_ref[:, DIM:DIM + 3]
    P = (pc[:, 0:1] * Wp1_ref[0:1, :] + pc[:, 1:2] * Wp1_ref[1:2, :]
         + pc[:, 2:3] * Wp1_ref[2:3, :])
    out_ref[...] = jnp.concatenate([v, kA, P], axis=1)


def _t5b_body(qpc_ref, spc_ref, out_ref, *, Qk):
    qx = qpc_ref[0, :, 0:1]
    qy = qpc_ref[0, :, 1:2]
    qz = qpc_ref[0, :, 2:3]
    sx = spc_ref[0, 0:1, :]
    sy = spc_ref[0, 1:2, :]
    sz = spc_ref[0, 2:3, :]
    sn = sx * sx + sy * sy + sz * sz
    qn = qx * qx + qy * qy + qz * qz
    qmat = jnp.concatenate([qx, qy, qz, jnp.ones_like(qx)], axis=1)
    smat = jnp.concatenate([-2.0 * sx, -2.0 * sy, -2.0 * sz, sn], axis=0)
    d = jnp.dot(qmat, smat, preferred_element_type=_f32) + qn
    d = jnp.maximum(d, 0.0)
    col = lax.broadcasted_iota(_i32, (Qk, N1), 1)
    key = lax.bitcast_convert_type(
        (lax.bitcast_convert_type(d, _i32) & ~jnp.int32(N1 - 1)) | col, _f32)
    boff = pl.program_id(0) * N1
    cols = []
    inf = jnp.float32(jnp.inf)
    for r in range(KNN):
        kmin = jnp.min(key, axis=1, keepdims=True)
        cols.append(lax.bitcast_convert_type(kmin, _i32) & jnp.int32(N1 - 1))
        key = jnp.where(key == kmin, inf, key)
    out_ref[0] = jnp.concatenate(cols, axis=1) + boff


def _sc_gather(table, idx, width, rows_per_worker):
    R = idx.shape[0]
    nchunk = rows_per_worker // 128
    idx2 = idx.reshape(R // 128, 128)
    mesh = plsc.VectorSubcoreMesh(core_axis_name="c", subcore_axis_name="s")

    @functools.partial(
        pl.kernel, mesh=mesh,
        out_type=jax.ShapeDtypeStruct((R, width), _f32),
        scratch_types=[
            pltpu.VMEM((nchunk, 128), _i32),
            pltpu.VMEM((128, width), _f32),
            pltpu.SemaphoreType.DMA,
        ],
    )
    def k(table_hbm, idx_hbm, out_hbm, idx_v, rows_v, sem):
        wid = lax.axis_index("s") * NC + lax.axis_index("c")
        base = wid * nchunk
        pltpu.sync_copy(idx_hbm.at[pl.ds(base, nchunk)], idx_v)
        for c in range(nchunk):
            pltpu.async_copy(table_hbm.at[idx_v.at[c]], rows_v, sem).wait()
            pltpu.sync_copy(rows_v,
                            out_hbm.at[pl.ds((base + c) * 128, 128)])

    return k(table, idx2)


def _t6_body(g_ref, Pq_ref, b1_ref, out_ref):
    pp = g_ref[0][:, :, DIM + AH:]
    pos_pre = pp - Pq_ref[0][:, None, :] + b1_ref[...]
    f = pos_pre.reshape(-1, PH)
    out_ref[0, 0, 0:1, :] = jnp.sum(f, axis=0, keepdims=True)
    out_ref[0, 0, 1:2, :] = jnp.sum(f * f, axis=0, keepdims=True)


def _stats_final_body(st_ref, g_ref, be_ref, out_ref):
    S = jnp.sum(st_ref[:, :, 0:1, :], axis=(0, 1))
    SS = jnp.sum(st_ref[:, :, 1:2, :], axis=(0, 1))
    mean = S / CNT
    var = SS / CNT - mean * mean
    scale = g_ref[...] * lax.rsqrt(var + EPS)
    out_ref[0:1, :] = scale
    out_ref[1:2, :] = be_ref[...] - mean * scale


def _pos_hidden(pp, Pq, b1, s1):
    pos_pre = pp - Pq[:, None, :] + b1
    return jnp.maximum(pos_pre * s1[0:1, :] + s1[1:2, :], 0.0)


def _t7_body(g_ref, Pq_ref, qA_ref, s1_ref, b1_ref, W2_ref, b2_ref,
             WA_ref, ab1_ref, out_ref):
    g = g_ref[0]
    kA = g[:, :, DIM:DIM + AH]
    posH = _pos_hidden(g[:, :, DIM + AH:], Pq_ref[0], b1_ref[...], s1_ref[...])
    W2A = jnp.dot(W2_ref[...], WA_ref[...], preferred_element_type=_f32)
    b2A = jnp.dot(b2_ref[...], WA_ref[...], preferred_element_type=_f32)
    Q = posH.shape[0]
    posA = jnp.dot(posH.reshape(-1, PH), W2A,
                   preferred_element_type=_f32).reshape(Q, KNN, AH)
    attn_pre = (kA - qA_ref[0][:, None, :] + posA + b2A
                + ab1_ref[...])
    f = attn_pre.reshape(-1, AH)
    out_ref[0, 0, 0:1, :] = jnp.sum(f, axis=0, keepdims=True)
    out_ref[0, 0, 1:2, :] = jnp.sum(f * f, axis=0, keepdims=True)


def _t8_body(g_ref, Pq_ref, qA_ref, s1_ref, s2_ref,
             resid_ref, b1_ref, W2_ref, b2_ref, WA_ref, ab1_ref,
             aW2_ref, ab2_ref, Wo1_ref, bo1_ref, Wo2_ref, bo2_ref,
             dec_ref, enc_ref):
    g = g_ref[0]
    vv = g[:, :, :DIM]
    kA = g[:, :, DIM:DIM + AH]
    posH = _pos_hidden(g[:, :, DIM + AH:], Pq_ref[0], b1_ref[...], s1_ref[...])
    Q = posH.shape[0]
    posHf = posH.reshape(-1, PH)
    pos = (jnp.dot(posHf, W2_ref[...], preferred_element_type=_f32)
           .reshape(Q, KNN, DIM) + b2_ref[...])
    W2A = jnp.dot(W2_ref[...], WA_ref[...], preferred_element_type=_f32)
    b2A = jnp.dot(b2_ref[...], WA_ref[...], preferred_element_type=_f32)
    posA = jnp.dot(posHf, W2A, preferred_element_type=_f32).reshape(Q, KNN, AH)
    attn_pre = (kA - qA_ref[0][:, None, :] + posA + b2A
                + ab1_ref[...])
    h2 = jnp.maximum(attn_pre * s2_ref[0:1, :] + s2_ref[1:2, :], 0.0)
    attn = (jnp.dot(h2.reshape(-1, AH), aW2_ref[...],
                    preferred_element_type=_f32).reshape(Q, KNN, DIM)
            + ab2_ref[...])
    mx = jnp.max(attn, axis=1, keepdims=True)
    e = jnp.exp(attn - mx)
    a = e / jnp.sum(e, axis=1, keepdims=True)
    o = jnp.sum((vv + pos) * a, axis=1)
    dec_ref[0] = (jnp.dot(o, Wo1_ref[...], preferred_element_type=_f32)
                  + bo1_ref[...] + resid_ref[0])
    enc_ref[0] = (jnp.dot(o, Wo2_ref[...], preferred_element_type=_f32)
                  + bo2_ref[...])


def _full(a):
    return pl.BlockSpec(a.shape, lambda *_: (0,) * a.ndim)


def kernel(dec_x, dec_pc, enc_x, enc_pc, W_pre1, b_pre1, W_pre2, b_pre2,
           Wq, bq, Wk, bk, Wv, bv, pos_W1, pos_b1, pos_g, pos_be,
           pos_W2, pos_b2, attn_W1, attn_b1, attn_g, attn_be,
           attn_W2, attn_b2, W_post1, b_post1, W_post2, b_post2):
    r2 = lambda v: v.reshape(1, -1)
    b_pre1r, bqr, bkr, bvr = r2(b_pre1), r2(bq), r2(bk), r2(bv)
    b_pre2r, pos_b1r, pos_b2r = r2(b_pre2), r2(pos_b1), r2(pos_b2)
    attn_b1r, attn_b2r = r2(attn_b1), r2(attn_b2)
    b_post1r, b_post2r = r2(b_post1), r2(b_post2)
    pos_gr, pos_ber, attn_gr, attn_ber = (r2(pos_g), r2(pos_be),
                                          r2(attn_g), r2(attn_be))

    Q1 = 256
    dxp, qA, Pq = pl.pallas_call(
        _t1_body,
        grid=(B, N1 // Q1),
        in_specs=[
            pl.BlockSpec((1, Q1, IN1), lambda b, n: (b, n, 0)),
            pl.BlockSpec((1, Q1, 3), lambda b, n: (b, n, 0)),
            _full(W_pre1), _full(b_pre1r), _full(Wq), _full(bqr),
            _full(attn_W1), _full(pos_W1),
        ],
        out_specs=[
            pl.BlockSpec((1, Q1, DIM), lambda b, n: (b, n, 0)),
            pl.BlockSpec((1, Q1, AH), lambda b, n: (b, n, 0)),
            pl.BlockSpec((1, Q1, PH), lambda b, n: (b, n, 0)),
        ],
        out_shape=[
            jax.ShapeDtypeStruct((B, N1, DIM), _f32),
            jax.ShapeDtypeStruct((B, N1, AH), _f32),
            jax.ShapeDtypeStruct((B, N1, PH), _f32),
        ],
    )(dec_x, dec_pc, W_pre1, b_pre1r, Wq, bqr, attn_W1, pos_W1)

    Q2 = 256
    exp_ = pl.pallas_call(
        _t2_body,
        grid=(B, N2 // Q2),
        in_specs=[
            pl.BlockSpec((1, Q2, IN2), lambda b, n: (b, n, 0)),
            _full(W_pre2), _full(b_pre2r),
        ],
        out_specs=pl.BlockSpec((1, Q2, DIM), lambda b, n: (b, n, 0)),
        out_shape=jax.ShapeDtypeStruct((B, N2, DIM), _f32),
    )(enc_x, W_pre2, b_pre2r)

    pc_cat = jnp.concatenate([dec_pc, enc_pc], axis=1)
    xflat = jnp.concatenate(
        [jnp.concatenate([dxp, exp_], axis=1).reshape(B * M, DIM),
         pc_cat.reshape(B * M, 3)], axis=1)
    xflat = jnp.pad(xflat, ((0, 0), (0, 128 - DIM - 3)))
    pc_t = jnp.transpose(pc_cat, (2, 0, 1))

    fps_gidx = pl.pallas_call(
        _fps_body,
        in_specs=[_full(pc_t)],
        out_specs=pl.BlockSpec((B, N1), lambda: (0, 0)),
        out_shape=jax.ShapeDtypeStruct((B, N1), _i32),
    )(pc_t)

    x_s = _sc_gather(xflat, fps_gidx.reshape(-1), 128, (B * N1) // NW)

    TW = DIM + AH + PH
    Qa = 1024
    tbl = pl.pallas_call(
        _t5a_body,
        grid=((B * N1) // Qa,),
        in_specs=[
            pl.BlockSpec((Qa, 128), lambda i: (i, 0)),
            _full(Wv), _full(bvr), _full(Wk), _full(bkr),
            _full(attn_W1), _full(pos_W1),
        ],
        out_specs=pl.BlockSpec((Qa, TW), lambda i: (i, 0)),
        out_shape=jax.ShapeDtypeStruct((B * N1, TW), _f32),
    )(x_s, Wv, bvr, Wk, bkr, attn_W1, pos_W1)

    Qk = 128
    pcs_b = jnp.transpose(
        x_s[:, DIM:DIM + 3].reshape(B, N1, 3), (0, 2, 1))
    knn_gidx = pl.pallas_call(
        functools.partial(_t5b_body, Qk=Qk),
        grid=(B, N1 // Qk),
        in_specs=[
            pl.BlockSpec((1, Qk, 3), lambda b, n: (b, n, 0)),
            pl.BlockSpec((1, 3, N1), lambda b, n: (b, 0, 0)),
        ],
        out_specs=pl.BlockSpec((1, Qk, KNN), lambda b, n: (b, n, 0)),
        out_shape=jax.ShapeDtypeStruct((B, N1, KNN), _i32),
    )(dec_pc, pcs_b)

    G = _sc_gather(tbl, knn_gidx.reshape(-1), TW, (B * N1 * KNN) // NW)
    G4 = G.reshape(B, N1, KNN, TW)

    Qs = 128
    nq = N1 // Qs
    st1 = pl.pallas_call(
        _t6_body,
        grid=(B, nq),
        in_specs=[
            pl.BlockSpec((1, Qs, KNN, TW), lambda b, n: (b, n, 0, 0)),
            pl.BlockSpec((1, Qs, PH), lambda b, n: (b, n, 0)),
            _full(pos_b1r),
        ],
        out_specs=pl.BlockSpec((1, 1, 2, PH), lambda b, n: (b, n, 0, 0)),
        out_shape=jax.ShapeDtypeStruct((B, nq, 2, PH), _f32),
    )(G4, Pq, pos_b1r)
    s1 = pl.pallas_call(
        _stats_final_body,
        in_specs=[_full(st1), _full(pos_gr), _full(pos_ber)],
        out_specs=pl.BlockSpec((2, PH), lambda: (0, 0)),
        out_shape=jax.ShapeDtypeStruct((2, PH), _f32),
    )(st1, pos_gr, pos_ber)

    st2 = pl.pallas_call(
        _t7_body,
        grid=(B, nq),
        in_specs=[
            pl.BlockSpec((1, Qs, KNN, TW), lambda b, n: (b, n, 0, 0)),
            pl.BlockSpec((1, Qs, PH), lambda b, n: (b, n, 0)),
            pl.BlockSpec((1, Qs, AH), lambda b, n: (b, n, 0)),
            _full(s1), _full(pos_b1r), _full(pos_W2), _full(pos_b2r),
            _full(attn_W1), _full(attn_b1r),
        ],
        out_specs=pl.BlockSpec((1, 1, 2, AH), lambda b, n: (b, n, 0, 0)),
        out_shape=jax.ShapeDtypeStruct((B, nq, 2, AH), _f32),
    )(G4, Pq, qA, s1, pos_b1r, pos_W2, pos_b2r, attn_W1, attn_b1r)
    s2 = pl.pallas_call(
        _stats_final_body,
        in_specs=[_full(st2), _full(attn_gr), _full(attn_ber)],
        out_specs=pl.BlockSpec((2, AH), lambda: (0, 0)),
        out_shape=jax.ShapeDtypeStruct((2, AH), _f32),
    )(st2, attn_gr, attn_ber)

    Qf = 128
    dec_out, enc_out = pl.pallas_call(
        _t8_body,
        grid=(B, N1 // Qf),
        in_specs=[
            pl.BlockSpec((1, Qf, KNN, TW), lambda b, n: (b, n, 0, 0)),
            pl.BlockSpec((1, Qf, PH), lambda b, n: (b, n, 0)),
            pl.BlockSpec((1, Qf, AH), lambda b, n: (b, n, 0)),
            _full(s1), _full(s2),
            pl.BlockSpec((1, Qf, IN1), lambda b, n: (b, n, 0)),
            _full(pos_b1r), _full(pos_W2), _full(pos_b2r), _full(attn_W1),
            _full(attn_b1r), _full(attn_W2), _full(attn_b2r),
            _full(W_post1), _full(b_post1r), _full(W_post2), _full(b_post2r),
        ],
        out_specs=[
            pl.BlockSpec((1, Qf, IN1), lambda b, n: (b, n, 0)),
            pl.BlockSpec((1, Qf, IN2), lambda b, n: (b, n, 0)),
        ],
        out_shape=[
            jax.ShapeDtypeStruct((B, N1, IN1), _f32),
            jax.ShapeDtypeStruct((B, N1, IN2), _f32),
        ],
    )(G4, Pq, qA, s1, s2, dec_x, pos_b1r, pos_W2, pos_b2r,
      attn_W1, attn_b1r, attn_W2, attn_b2r, W_post1, b_post1r,
      W_post2, b_post2r)

    return (dec_out, dec_pc, enc_out, enc_pc)
